# Initial kernel scaffold; baseline (speedup 1.0000x reference)
import jax
import jax.numpy as jnp
from jax.experimental import pallas as pl


def kernel(interp_xs, interp_ys, xs, ys, zs):
    raise NotImplementedError("write your pallas kernel here")


# speedup vs baseline: 503.6389x; 503.6389x over previous
import functools

import jax
import jax.numpy as jnp
from jax import lax
from jax.experimental import pallas as pl
from jax.experimental.pallas import tpu as pltpu
from jax.experimental.pallas import tpu_sc as plsc

GRID = 4096
NPTS = 4194304
NC = 2
NS = 16
NW = NC * NS
PTS_PER_TILE = NPTS // NW
B = 2048
NB = B // 128
NCHUNK = PTS_PER_TILE // B
LOG2_GRID = 12
SEARCH_STEPS = 13


def _searchsorted_left(table_ref, x):
    lo = jnp.zeros((16,), jnp.int32)
    hi = jnp.full((16,), GRID, jnp.int32)
    for _ in range(SEARCH_STEPS):
        mid = jnp.minimum((lo + hi) >> 1, GRID - 1)
        v = plsc.load_gather(table_ref, [mid])
        pred = v < x
        lo = jnp.where(pred, mid + 1, lo)
        hi = jnp.where(pred, hi, mid)
    return lo


def _axis_dists(table_ref, x):
    ins = _searchsorted_left(table_ref, x)
    r = jnp.minimum(ins, GRID - 1)
    l = jnp.maximum(r - 1, 0)
    vl = plsc.load_gather(table_ref, [l])
    vr = plsc.load_gather(table_ref, [r])
    dl = jnp.maximum(x - vl, 0.0)
    dr = jnp.maximum(vr - x, 0.0)
    both_zero = (dl == 0.0) & (dr == 0.0)
    dl = jnp.where(both_zero, 1.0, dl)
    dr = jnp.where(both_zero, 1.0, dr)
    return l, r, dl, dr


def _body(xq_hbm, yq_hbm, xs_hbm, ys_hbm, zs_hbm, out_hbm,
          xs_t, ys_t, xq, yq, wbuf, idx, cbuf, outb, gsem):
    wid = lax.axis_index("s") * NC + lax.axis_index("c")
    base = wid * PTS_PER_TILE

    pltpu.sync_copy(xs_hbm, xs_t)
    pltpu.sync_copy(ys_hbm, ys_t)

    def pass1_row(j, _):
        for k in range(8):
            off = j * 128 + k * 16
            x = xq[pl.ds(off, 16)]
            y = yq[pl.ds(off, 16)]
            ixl, ixr, dw, de = _axis_dists(xs_t, x)
            iyl, iyr, dsn, dn = _axis_dists(ys_t, y)
            west = ixl << LOG2_GRID
            east = ixr << LOG2_GRID
            idx[0, j, pl.ds(k * 16, 16)] = west + iyl
            idx[1, j, pl.ds(k * 16, 16)] = west + iyr
            idx[2, j, pl.ds(k * 16, 16)] = east + iyr
            idx[3, j, pl.ds(k * 16, 16)] = east + iyl
            rden = 1.0 / ((dw + de) * (dn + dsn))
            wbuf[0, pl.ds(off, 16)] = dn * de * rden
            wbuf[1, pl.ds(off, 16)] = dsn * de * rden
            wbuf[2, pl.ds(off, 16)] = dsn * dw * rden
            wbuf[3, pl.ds(off, 16)] = dn * dw * rden
        for c in range(4):
            pltpu.async_copy(zs_hbm.at[idx.at[c, j]], cbuf.at[c, j], gsem)
        return 0

    def drain_row(j, _):
        for c in range(4):
            pltpu.make_async_copy(zs_hbm.at[idx.at[c, j]], cbuf.at[c, j],
                                  gsem).wait()
        return 0

    def pass2_row(j, _):
        for k in range(8):
            off = j * 128 + k * 16
            acc = cbuf[0, j, pl.ds(k * 16, 16)] * wbuf[0, pl.ds(off, 16)]
            acc += cbuf[1, j, pl.ds(k * 16, 16)] * wbuf[1, pl.ds(off, 16)]
            acc += cbuf[2, j, pl.ds(k * 16, 16)] * wbuf[2, pl.ds(off, 16)]
            acc += cbuf[3, j, pl.ds(k * 16, 16)] * wbuf[3, pl.ds(off, 16)]
            outb[pl.ds(off, 16)] = acc
        return 0

    def chunk(g, _):
        off = base + g * B
        pltpu.sync_copy(xq_hbm.at[pl.ds(off, B)], xq)
        pltpu.sync_copy(yq_hbm.at[pl.ds(off, B)], yq)
        lax.fori_loop(0, NB, pass1_row, 0)
        lax.fori_loop(0, NB, drain_row, 0)
        lax.fori_loop(0, NB, pass2_row, 0)
        pltpu.sync_copy(outb, out_hbm.at[pl.ds(off, B)])
        return 0

    lax.fori_loop(0, NCHUNK, chunk, 0)


@jax.jit
def _interp(interp_xs, interp_ys, xs, ys, zs_flat):
    mesh = plsc.VectorSubcoreMesh(core_axis_name="c", subcore_axis_name="s")
    fn = pl.kernel(
        _body,
        out_type=jax.ShapeDtypeStruct((NPTS,), jnp.float32),
        mesh=mesh,
        compiler_params=pltpu.CompilerParams(needs_layout_passes=False),
        scratch_types=[
            pltpu.VMEM((GRID,), jnp.float32),
            pltpu.VMEM((GRID,), jnp.float32),
            pltpu.VMEM((B,), jnp.float32),
            pltpu.VMEM((B,), jnp.float32),
            pltpu.VMEM((4, B), jnp.float32),
            pltpu.VMEM((4, NB, 128), jnp.int32),
            pltpu.VMEM((4, NB, 128), jnp.float32),
            pltpu.VMEM((B,), jnp.float32),
            pltpu.SemaphoreType.DMA,
        ],
    )
    return fn(interp_xs, interp_ys, xs, ys, zs_flat)


def kernel(interp_xs, interp_ys, xs, ys, zs):
    return _interp(interp_xs, interp_ys, xs, ys, zs.reshape(-1))


# speedup vs baseline: 566.2376x; 1.1243x over previous
import jax
import jax.numpy as jnp
from jax import lax
from jax.experimental import pallas as pl
from jax.experimental.pallas import tpu as pltpu
from jax.experimental.pallas import tpu_sc as plsc

GRID = 4096
NPTS = 4194304
NC = 2
NS = 16
NW = NC * NS
PTS_PER_TILE = NPTS // NW
B = 2048
NB = B // 128
NCHUNK = PTS_PER_TILE // B
LOG2_GRID = 12
SEARCH_STEPS = 13
NBIN = 4096
TPAD = NBIN + 16
DELTA = 1000.0 / NBIN
INV_DELTA = NBIN / 1000.0


def _searchsorted_full(table_ref, x):
    lo = jnp.zeros((16,), jnp.int32)
    hi = jnp.full((16,), GRID, jnp.int32)
    for _ in range(SEARCH_STEPS):
        mid = jnp.minimum((lo + hi) >> 1, GRID - 1)
        v = plsc.load_gather(table_ref, [mid])
        pred = v < x
        lo = jnp.where(pred, mid + 1, lo)
        hi = jnp.where(pred, hi, mid)
    return lo


def _bin_of(x):
    b = jnp.minimum((x * INV_DELTA).astype(jnp.int32), NBIN - 1)
    b = jnp.where(x < b.astype(jnp.float32) * DELTA, b - 1, b)
    b = jnp.where(x >= (b + 1).astype(jnp.float32) * DELTA, b + 1, b)
    return b


def _build_accel(table_ref, t_ref):
    lanes = lax.iota(jnp.int32, 16)

    def body(i, _):
        q = (lanes + i * 16).astype(jnp.float32) * DELTA
        t_ref[pl.ds(i * 16, 16)] = _searchsorted_full(table_ref, q)
        return 0

    lax.fori_loop(0, TPAD // 16, body, 0)


def _axis_dists(table_ref, t_ref, x):
    b = _bin_of(x)
    lo = plsc.load_gather(t_ref, [b])
    hi = plsc.load_gather(t_ref, [b + 1])

    def cond(carry):
        lo, hi = carry
        return jnp.any(lo < hi)

    def step(carry):
        lo, hi = carry
        mid = jnp.minimum((lo + hi) >> 1, GRID - 1)
        v = plsc.load_gather(table_ref, [mid])
        pred = v < x
        return (jnp.where(pred, mid + 1, lo), jnp.where(pred, hi, mid))

    ins, _ = lax.while_loop(cond, step, (lo, hi))
    r = jnp.minimum(ins, GRID - 1)
    l = jnp.maximum(r - 1, 0)
    vl = plsc.load_gather(table_ref, [l])
    vr = plsc.load_gather(table_ref, [r])
    dl = jnp.maximum(x - vl, 0.0)
    dr = jnp.maximum(vr - x, 0.0)
    both_zero = (dl == 0.0) & (dr == 0.0)
    dl = jnp.where(both_zero, 1.0, dl)
    dr = jnp.where(both_zero, 1.0, dr)
    return l, r, dl, dr


def _body(xq_hbm, yq_hbm, xs_hbm, ys_hbm, zs_hbm, out_hbm,
          xs_t, ys_t, tx, ty, xq, yq, wbuf, idx, cbuf, outb, gsem):
    wid = lax.axis_index("s") * NC + lax.axis_index("c")
    base = wid * PTS_PER_TILE

    pltpu.sync_copy(xs_hbm, xs_t)
    pltpu.sync_copy(ys_hbm, ys_t)
    _build_accel(xs_t, tx)
    _build_accel(ys_t, ty)

    def pass1_row(j, _):
        for k in range(8):
            off = j * 128 + k * 16
            x = xq[pl.ds(off, 16)]
            y = yq[pl.ds(off, 16)]
            ixl, ixr, dw, de = _axis_dists(xs_t, tx, x)
            iyl, iyr, dsn, dn = _axis_dists(ys_t, ty, y)
            west = ixl << LOG2_GRID
            east = ixr << LOG2_GRID
            idx[0, j, pl.ds(k * 16, 16)] = west + iyl
            idx[1, j, pl.ds(k * 16, 16)] = west + iyr
            idx[2, j, pl.ds(k * 16, 16)] = east + iyr
            idx[3, j, pl.ds(k * 16, 16)] = east + iyl
            rden = 1.0 / ((dw + de) * (dn + dsn))
            wbuf[0, pl.ds(off, 16)] = dn * de * rden
            wbuf[1, pl.ds(off, 16)] = dsn * de * rden
            wbuf[2, pl.ds(off, 16)] = dsn * dw * rden
            wbuf[3, pl.ds(off, 16)] = dn * dw * rden
        for c in range(4):
            pltpu.async_copy(zs_hbm.at[idx.at[c, j]], cbuf.at[c, j], gsem)
        return 0

    def drain_row(j, _):
        for c in range(4):
            pltpu.make_async_copy(zs_hbm.at[idx.at[c, j]], cbuf.at[c, j],
                                  gsem).wait()
        return 0

    def pass2_row(j, _):
        for k in range(8):
            off = j * 128 + k * 16
            acc = cbuf[0, j, pl.ds(k * 16, 16)] * wbuf[0, pl.ds(off, 16)]
            acc += cbuf[1, j, pl.ds(k * 16, 16)] * wbuf[1, pl.ds(off, 16)]
            acc += cbuf[2, j, pl.ds(k * 16, 16)] * wbuf[2, pl.ds(off, 16)]
            acc += cbuf[3, j, pl.ds(k * 16, 16)] * wbuf[3, pl.ds(off, 16)]
            outb[pl.ds(off, 16)] = acc
        return 0

    def chunk(g, _):
        off = base + g * B
        pltpu.sync_copy(xq_hbm.at[pl.ds(off, B)], xq)
        pltpu.sync_copy(yq_hbm.at[pl.ds(off, B)], yq)
        lax.fori_loop(0, NB, pass1_row, 0)
        lax.fori_loop(0, NB, drain_row, 0)
        lax.fori_loop(0, NB, pass2_row, 0)
        pltpu.sync_copy(outb, out_hbm.at[pl.ds(off, B)])
        return 0

    lax.fori_loop(0, NCHUNK, chunk, 0)


@jax.jit
def _interp(interp_xs, interp_ys, xs, ys, zs_flat):
    mesh = plsc.VectorSubcoreMesh(core_axis_name="c", subcore_axis_name="s")
    fn = pl.kernel(
        _body,
        out_type=jax.ShapeDtypeStruct((NPTS,), jnp.float32),
        mesh=mesh,
        compiler_params=pltpu.CompilerParams(needs_layout_passes=False),
        scratch_types=[
            pltpu.VMEM((GRID,), jnp.float32),
            pltpu.VMEM((GRID,), jnp.float32),
            pltpu.VMEM((TPAD,), jnp.int32),
            pltpu.VMEM((TPAD,), jnp.int32),
            pltpu.VMEM((B,), jnp.float32),
            pltpu.VMEM((B,), jnp.float32),
            pltpu.VMEM((4, B), jnp.float32),
            pltpu.VMEM((4, NB, 128), jnp.int32),
            pltpu.VMEM((4, NB, 128), jnp.float32),
            pltpu.VMEM((B,), jnp.float32),
            pltpu.SemaphoreType.DMA,
        ],
    )
    return fn(interp_xs, interp_ys, xs, ys, zs_flat)


def kernel(interp_xs, interp_ys, xs, ys, zs):
    return _interp(interp_xs, interp_ys, xs, ys, zs.reshape(-1))


# speedup vs baseline: 669.2410x; 1.1819x over previous
import functools

import jax
import jax.numpy as jnp
from jax import lax
from jax.experimental import pallas as pl
from jax.experimental.pallas import tpu as pltpu
from jax.experimental.pallas import tpu_sc as plsc

GRID = 4096
NPTS = 4194304
NC = 2
NS = 16
NW = NC * NS
PTS_PER_TILE = NPTS // NW
B = 2048
NB = B // 128
NCHUNK = PTS_PER_TILE // B
LOG2_GRID = 12
SEARCH_STEPS = 13


def _axis_dists_multi(tabs, vals):
    n = len(vals)
    lo = [jnp.zeros((16,), jnp.int32) for _ in range(n)]
    hi = [jnp.full((16,), GRID, jnp.int32) for _ in range(n)]
    for _ in range(SEARCH_STEPS):
        mid = [jnp.minimum((lo[c] + hi[c]) >> 1, GRID - 1) for c in range(n)]
        v = [plsc.load_gather(tabs[c], [mid[c]]) for c in range(n)]
        pred = [v[c] < vals[c] for c in range(n)]
        lo = [jnp.where(pred[c], mid[c] + 1, lo[c]) for c in range(n)]
        hi = [jnp.where(pred[c], hi[c], mid[c]) for c in range(n)]
    r = [jnp.minimum(lo[c], GRID - 1) for c in range(n)]
    l = [jnp.maximum(r[c] - 1, 0) for c in range(n)]
    vl = [plsc.load_gather(tabs[c], [l[c]]) for c in range(n)]
    vr = [plsc.load_gather(tabs[c], [r[c]]) for c in range(n)]
    out = []
    for c in range(n):
        dl = jnp.maximum(vals[c] - vl[c], 0.0)
        dr = jnp.maximum(vr[c] - vals[c], 0.0)
        both_zero = (dl == 0.0) & (dr == 0.0)
        dl = jnp.where(both_zero, 1.0, dl)
        dr = jnp.where(both_zero, 1.0, dr)
        out.append((l[c], r[c], dl, dr))
    return out


def _body(xq_hbm, yq_hbm, xs_hbm, ys_hbm, zs_hbm, out_hbm,
          xs_t, ys_t, xq, yq, wbuf, idx, cbuf, outb, gsem):
    wid = lax.axis_index("s") * NC + lax.axis_index("c")
    base = wid * PTS_PER_TILE

    pltpu.sync_copy(xs_hbm, xs_t)
    pltpu.sync_copy(ys_hbm, ys_t)

    def pass1_row(j, _):
        for h in range(2):
            ks = [4 * h + t for t in range(4)]
            xv = [xq[pl.ds(j * 128 + k * 16, 16)] for k in ks]
            yv = [yq[pl.ds(j * 128 + k * 16, 16)] for k in ks]
            res = _axis_dists_multi([xs_t] * 4 + [ys_t] * 4, xv + yv)
            for t, k in enumerate(ks):
                off = j * 128 + k * 16
                ixl, ixr, dw, de = res[t]
                iyl, iyr, dsn, dn = res[4 + t]
                west = ixl << LOG2_GRID
                east = ixr << LOG2_GRID
                idx[0, j, pl.ds(k * 16, 16)] = west + iyl
                idx[1, j, pl.ds(k * 16, 16)] = west + iyr
                idx[2, j, pl.ds(k * 16, 16)] = east + iyr
                idx[3, j, pl.ds(k * 16, 16)] = east + iyl
                rden = 1.0 / ((dw + de) * (dn + dsn))
                wbuf[0, pl.ds(off, 16)] = dn * de * rden
                wbuf[1, pl.ds(off, 16)] = dsn * de * rden
                wbuf[2, pl.ds(off, 16)] = dsn * dw * rden
                wbuf[3, pl.ds(off, 16)] = dn * dw * rden
        for c in range(4):
            pltpu.async_copy(zs_hbm.at[idx.at[c, j]], cbuf.at[c, j], gsem)
        return 0

    def drain_row(j, _):
        for c in range(4):
            pltpu.make_async_copy(zs_hbm.at[idx.at[c, j]], cbuf.at[c, j],
                                  gsem).wait()
        return 0

    def pass2_row(j, _):
        for k in range(8):
            off = j * 128 + k * 16
            acc = cbuf[0, j, pl.ds(k * 16, 16)] * wbuf[0, pl.ds(off, 16)]
            acc += cbuf[1, j, pl.ds(k * 16, 16)] * wbuf[1, pl.ds(off, 16)]
            acc += cbuf[2, j, pl.ds(k * 16, 16)] * wbuf[2, pl.ds(off, 16)]
            acc += cbuf[3, j, pl.ds(k * 16, 16)] * wbuf[3, pl.ds(off, 16)]
            outb[pl.ds(off, 16)] = acc
        return 0

    def chunk(g, _):
        off = base + g * B
        pltpu.sync_copy(xq_hbm.at[pl.ds(off, B)], xq)
        pltpu.sync_copy(yq_hbm.at[pl.ds(off, B)], yq)
        lax.fori_loop(0, NB, pass1_row, 0)
        lax.fori_loop(0, NB, drain_row, 0)
        lax.fori_loop(0, NB, pass2_row, 0)
        pltpu.sync_copy(outb, out_hbm.at[pl.ds(off, B)])
        return 0

    lax.fori_loop(0, NCHUNK, chunk, 0)


@jax.jit
def _interp(interp_xs, interp_ys, xs, ys, zs_flat):
    mesh = plsc.VectorSubcoreMesh(core_axis_name="c", subcore_axis_name="s")
    fn = pl.kernel(
        _body,
        out_type=jax.ShapeDtypeStruct((NPTS,), jnp.float32),
        mesh=mesh,
        compiler_params=pltpu.CompilerParams(needs_layout_passes=False),
        scratch_types=[
            pltpu.VMEM((GRID,), jnp.float32),
            pltpu.VMEM((GRID,), jnp.float32),
            pltpu.VMEM((B,), jnp.float32),
            pltpu.VMEM((B,), jnp.float32),
            pltpu.VMEM((4, B), jnp.float32),
            pltpu.VMEM((4, NB, 128), jnp.int32),
            pltpu.VMEM((4, NB, 128), jnp.float32),
            pltpu.VMEM((B,), jnp.float32),
            pltpu.SemaphoreType.DMA,
        ],
    )
    return fn(interp_xs, interp_ys, xs, ys, zs_flat)


def kernel(interp_xs, interp_ys, xs, ys, zs):
    return _interp(interp_xs, interp_ys, xs, ys, zs.reshape(-1))


# speedup vs baseline: 728.1364x; 1.0880x over previous
import jax
import jax.numpy as jnp
from jax import lax
from jax.experimental import pallas as pl
from jax.experimental.pallas import tpu as pltpu
from jax.experimental.pallas import tpu_sc as plsc

GRID = 4096
NPTS = 4194304
NC = 2
NS = 16
NW = NC * NS
PTS_PER_TILE = NPTS // NW
B = 2048
NB = B // 128
NCHUNK = PTS_PER_TILE // B
LOG2_GRID = 12
SEARCH_STEPS = 13


def _axis_dists_multi(tabs, vals):
    n = len(vals)
    lo = [jnp.zeros((16,), jnp.int32) for _ in range(n)]
    hi = [jnp.full((16,), GRID, jnp.int32) for _ in range(n)]
    for _ in range(SEARCH_STEPS):
        mid = [jnp.minimum((lo[c] + hi[c]) >> 1, GRID - 1) for c in range(n)]
        v = [plsc.load_gather(tabs[c], [mid[c]]) for c in range(n)]
        pred = [v[c] < vals[c] for c in range(n)]
        lo = [jnp.where(pred[c], mid[c] + 1, lo[c]) for c in range(n)]
        hi = [jnp.where(pred[c], hi[c], mid[c]) for c in range(n)]
    r = [jnp.minimum(lo[c], GRID - 1) for c in range(n)]
    l = [jnp.maximum(r[c] - 1, 0) for c in range(n)]
    vl = [plsc.load_gather(tabs[c], [l[c]]) for c in range(n)]
    vr = [plsc.load_gather(tabs[c], [r[c]]) for c in range(n)]
    out = []
    for c in range(n):
        dl = jnp.maximum(vals[c] - vl[c], 0.0)
        dr = jnp.maximum(vr[c] - vals[c], 0.0)
        both_zero = (dl == 0.0) & (dr == 0.0)
        dl = jnp.where(both_zero, 1.0, dl)
        dr = jnp.where(both_zero, 1.0, dr)
        out.append((l[c], r[c], dl, dr))
    return out


def _body(xq_hbm, yq_hbm, xs_hbm, ys_hbm, zs_hbm, out_hbm, xs_t, ys_t,
          xqA, yqA, wA, idxA, cA, outA,
          xqB, yqB, wB, idxB, cB, outB,
          insemA, insemB, gsemA, gsemB, osemA, osemB):
    wid = lax.axis_index("s") * NC + lax.axis_index("c")
    base = wid * PTS_PER_TILE

    pltpu.sync_copy(xs_hbm, xs_t)
    pltpu.sync_copy(ys_hbm, ys_t)

    bufA = (xqA, yqA, wA, idxA, cA, outA, insemA, gsemA, osemA)
    bufB = (xqB, yqB, wB, idxB, cB, outB, insemB, gsemB, osemB)

    def fetch(g, bs):
        xq, yq, insem = bs[0], bs[1], bs[6]
        off = base + g * B
        pltpu.async_copy(xq_hbm.at[pl.ds(off, B)], xq, insem)
        pltpu.async_copy(yq_hbm.at[pl.ds(off, B)], yq, insem)

    def wait_fetch(bs):
        xq, yq, insem = bs[0], bs[1], bs[6]
        pltpu.make_async_copy(xq_hbm.at[pl.ds(base, B)], xq, insem).wait()
        pltpu.make_async_copy(yq_hbm.at[pl.ds(base, B)], yq, insem).wait()

    def pass1(bs):
        xq, yq, wbuf, idx, cbuf, gsem = bs[0], bs[1], bs[2], bs[3], bs[4], bs[7]

        def row(j, _):
            for h in range(2):
                ks = [4 * h + t for t in range(4)]
                xv = [xq[pl.ds(j * 128 + k * 16, 16)] for k in ks]
                yv = [yq[pl.ds(j * 128 + k * 16, 16)] for k in ks]
                res = _axis_dists_multi([xs_t] * 4 + [ys_t] * 4, xv + yv)
                for t, k in enumerate(ks):
                    off = j * 128 + k * 16
                    ixl, ixr, dw, de = res[t]
                    iyl, iyr, dsn, dn = res[4 + t]
                    west = ixl << LOG2_GRID
                    east = ixr << LOG2_GRID
                    idx[0, j, pl.ds(k * 16, 16)] = west + iyl
                    idx[1, j, pl.ds(k * 16, 16)] = west + iyr
                    idx[2, j, pl.ds(k * 16, 16)] = east + iyr
                    idx[3, j, pl.ds(k * 16, 16)] = east + iyl
                    rden = 1.0 / ((dw + de) * (dn + dsn))
                    wbuf[0, pl.ds(off, 16)] = dn * de * rden
                    wbuf[1, pl.ds(off, 16)] = dsn * de * rden
                    wbuf[2, pl.ds(off, 16)] = dsn * dw * rden
                    wbuf[3, pl.ds(off, 16)] = dn * dw * rden
            for c in range(4):
                pltpu.async_copy(zs_hbm.at[idx.at[c, j]], cbuf.at[c, j], gsem)
            return 0

        lax.fori_loop(0, NB, row, 0)

    def drain_pass2(bs):
        wbuf, idx, cbuf, outb, gsem = bs[2], bs[3], bs[4], bs[5], bs[7]

        def dr(j, _):
            for c in range(4):
                pltpu.make_async_copy(zs_hbm.at[idx.at[c, j]],
                                      cbuf.at[c, j], gsem).wait()
            return 0

        def p2(j, _):
            for k in range(8):
                off = j * 128 + k * 16
                acc = cbuf[0, j, pl.ds(k * 16, 16)] * wbuf[0, pl.ds(off, 16)]
                acc += cbuf[1, j, pl.ds(k * 16, 16)] * wbuf[1, pl.ds(off, 16)]
                acc += cbuf[2, j, pl.ds(k * 16, 16)] * wbuf[2, pl.ds(off, 16)]
                acc += cbuf[3, j, pl.ds(k * 16, 16)] * wbuf[3, pl.ds(off, 16)]
                outb[pl.ds(off, 16)] = acc
            return 0

        lax.fori_loop(0, NB, dr, 0)
        lax.fori_loop(0, NB, p2, 0)

    def out_fire(g, bs):
        pltpu.async_copy(bs[5], out_hbm.at[pl.ds(base + g * B, B)], bs[8])

    def out_wait(bs):
        pltpu.make_async_copy(bs[5], out_hbm.at[pl.ds(base, B)], bs[8]).wait()

    def step(g, cur, nxt):
        wait_fetch(cur)

        @pl.when(g + 1 < NCHUNK)
        def _():
            fetch(g + 1, nxt)

        pass1(cur)

        @pl.when(g >= 1)
        def _():
            @pl.when(g >= 3)
            def _():
                out_wait(nxt)
            drain_pass2(nxt)
            out_fire(g - 1, nxt)

    fetch(0, bufA)

    def pair(g2, _):
        g = g2 * 2
        step(g, bufA, bufB)
        step(g + 1, bufB, bufA)
        return 0

    lax.fori_loop(0, NCHUNK // 2, pair, 0)

    out_wait(bufB)
    drain_pass2(bufB)
    out_fire(NCHUNK - 1, bufB)
    out_wait(bufA)
    out_wait(bufB)


@jax.jit
def _interp(interp_xs, interp_ys, xs, ys, zs_flat):
    mesh = plsc.VectorSubcoreMesh(core_axis_name="c", subcore_axis_name="s")
    dbl = [
        pltpu.VMEM((B,), jnp.float32),
        pltpu.VMEM((B,), jnp.float32),
        pltpu.VMEM((4, B), jnp.float32),
        pltpu.VMEM((4, NB, 128), jnp.int32),
        pltpu.VMEM((4, NB, 128), jnp.float32),
        pltpu.VMEM((B,), jnp.float32),
    ]
    fn = pl.kernel(
        _body,
        out_type=jax.ShapeDtypeStruct((NPTS,), jnp.float32),
        mesh=mesh,
        compiler_params=pltpu.CompilerParams(needs_layout_passes=False),
        scratch_types=(
            [pltpu.VMEM((GRID,), jnp.float32),
             pltpu.VMEM((GRID,), jnp.float32)]
            + dbl + dbl
            + [pltpu.SemaphoreType.DMA] * 6
        ),
    )
    return fn(interp_xs, interp_ys, xs, ys, zs_flat)


def kernel(interp_xs, interp_ys, xs, ys, zs):
    return _interp(interp_xs, interp_ys, xs, ys, zs.reshape(-1))


# speedup vs baseline: 1727.9349x; 2.3731x over previous
import jax
import jax.numpy as jnp
from jax import lax
from jax.experimental import pallas as pl
from jax.experimental.pallas import tpu as pltpu
from jax.experimental.pallas import tpu_sc as plsc

GRID = 4096
NPTS = 4194304
NC = 2
NS = 16
NW = NC * NS
PTS_PER_TILE = NPTS // NW
B = 2048
NB = B // 128
NCHUNK = PTS_PER_TILE // B
LOG2_GRID = 12
SEARCH_STEPS = 13
REP_LEVELS = 10
TAIL_STEPS = SEARCH_STEPS - REP_LEVELS
NPATH = 1 << REP_LEVELS
REP_CAPS = [max(1 << s, 16) for s in range(REP_LEVELS)]


def _build_axis(table_ref, reps, lo_e, hi_e, lo_o, hi_o):
    lanes = lax.iota(jnp.int32, 16)

    def init(i, _):
        lo_e[pl.ds(i * 16, 16)] = jnp.zeros((16,), jnp.int32)
        hi_e[pl.ds(i * 16, 16)] = jnp.full((16,), GRID, jnp.int32)
        lo_o[pl.ds(i * 16, 16)] = jnp.zeros((16,), jnp.int32)
        hi_o[pl.ds(i * 16, 16)] = jnp.full((16,), GRID, jnp.int32)
        return 0

    lax.fori_loop(0, NPATH // 16, init, 0)

    for s in range(REP_LEVELS):
        src_lo, src_hi = (lo_e, hi_e) if s % 2 == 0 else (lo_o, hi_o)
        dst_lo, dst_hi = (lo_o, hi_o) if s % 2 == 0 else (lo_e, hi_e)
        rep = reps[s]
        nvec = max(1, (1 << s) // 16)

        def bodyb(i, _, rep=rep, src_lo=src_lo, src_hi=src_hi,
                  dst_lo=dst_lo, dst_hi=dst_hi):
            lo = src_lo[pl.ds(i * 16, 16)]
            hi = src_hi[pl.ds(i * 16, 16)]
            mid = (lo + hi) >> 1
            v = plsc.load_gather(table_ref, [mid])
            for pl_ in range(16):
                bc = jnp.broadcast_to(v[pl_], (16,))
                rep[pl.ds(i * 256 + pl_ * 16, 16)] = bc
            p2 = (i * 16 + lanes) << 1
            plsc.store_scatter(dst_lo, [p2], lo)
            plsc.store_scatter(dst_hi, [p2], mid)
            plsc.store_scatter(dst_lo, [p2 + 1], mid + 1)
            plsc.store_scatter(dst_hi, [p2 + 1], hi)
            return 0

        lax.fori_loop(0, nvec, bodyb, 0)


def _axis_dists_multi(axes, vals):
    n = len(vals)
    lanes = lax.iota(jnp.int32, 16)
    p = [jnp.zeros((16,), jnp.int32) for _ in range(n)]
    for s in range(REP_LEVELS):
        v = [plsc.load_gather(axes[c][1][s], [(p[c] << 4) + lanes])
             for c in range(n)]
        pred = [v[c] < vals[c] for c in range(n)]
        p = [(p[c] << 1) + pred[c].astype(jnp.int32) for c in range(n)]
    lo = [plsc.load_gather(axes[c][2], [p[c]]) for c in range(n)]
    hi = [plsc.load_gather(axes[c][3], [p[c]]) for c in range(n)]
    for _ in range(TAIL_STEPS):
        mid = [jnp.minimum((lo[c] + hi[c]) >> 1, GRID - 1) for c in range(n)]
        v = [plsc.load_gather(axes[c][0], [mid[c]]) for c in range(n)]
        pred = [v[c] < vals[c] for c in range(n)]
        lo = [jnp.where(pred[c], mid[c] + 1, lo[c]) for c in range(n)]
        hi = [jnp.where(pred[c], hi[c], mid[c]) for c in range(n)]
    r = [jnp.minimum(lo[c], GRID - 1) for c in range(n)]
    l = [jnp.maximum(r[c] - 1, 0) for c in range(n)]
    vl = [plsc.load_gather(axes[c][0], [l[c]]) for c in range(n)]
    vr = [plsc.load_gather(axes[c][0], [r[c]]) for c in range(n)]
    out = []
    for c in range(n):
        dl = jnp.maximum(vals[c] - vl[c], 0.0)
        dr = jnp.maximum(vr[c] - vals[c], 0.0)
        both_zero = (dl == 0.0) & (dr == 0.0)
        dl = jnp.where(both_zero, 1.0, dl)
        dr = jnp.where(both_zero, 1.0, dr)
        out.append((l[c], r[c], dl, dr))
    return out


def _body(xq_hbm, yq_hbm, xs_hbm, ys_hbm, zs_hbm, out_hbm, *refs):
    xs_t, ys_t = refs[0], refs[1]
    repx = list(refs[2:12])
    repy = list(refs[12:22])
    lox_e, hix_e, lox_o, hix_o = refs[22:26]
    loy_e, hiy_e, loy_o, hiy_o = refs[26:30]
    xqA, yqA, wA, idxA, cA, outA = refs[30:36]
    xqB, yqB, wB, idxB, cB, outB = refs[36:42]
    insemA, insemB, gsemA, gsemB, osemA, osemB = refs[42:48]

    wid = lax.axis_index("s") * NC + lax.axis_index("c")
    base = wid * PTS_PER_TILE

    pltpu.sync_copy(xs_hbm, xs_t)
    pltpu.sync_copy(ys_hbm, ys_t)
    _build_axis(xs_t, repx, lox_e, hix_e, lox_o, hix_o)
    _build_axis(ys_t, repy, loy_e, hiy_e, loy_o, hiy_o)
    xaxis = (xs_t, repx, lox_e, hix_e)
    yaxis = (ys_t, repy, loy_e, hiy_e)

    bufA = (xqA, yqA, wA, idxA, cA, outA, insemA, gsemA, osemA)
    bufB = (xqB, yqB, wB, idxB, cB, outB, insemB, gsemB, osemB)

    def fetch(g, bs):
        xq, yq, insem = bs[0], bs[1], bs[6]
        off = base + g * B
        pltpu.async_copy(xq_hbm.at[pl.ds(off, B)], xq, insem)
        pltpu.async_copy(yq_hbm.at[pl.ds(off, B)], yq, insem)

    def wait_fetch(bs):
        xq, yq, insem = bs[0], bs[1], bs[6]
        pltpu.make_async_copy(xq_hbm.at[pl.ds(base, B)], xq, insem).wait()
        pltpu.make_async_copy(yq_hbm.at[pl.ds(base, B)], yq, insem).wait()

    def pass1(bs):
        xq, yq, wbuf, idx, cbuf, gsem = (bs[0], bs[1], bs[2], bs[3], bs[4],
                                         bs[7])

        def row(j, _):
            for h in range(2):
                ks = [4 * h + t for t in range(4)]
                xv = [xq[pl.ds(j * 128 + k * 16, 16)] for k in ks]
                yv = [yq[pl.ds(j * 128 + k * 16, 16)] for k in ks]
                res = _axis_dists_multi([xaxis] * 4 + [yaxis] * 4, xv + yv)
                for t, k in enumerate(ks):
                    off = j * 128 + k * 16
                    ixl, ixr, dw, de = res[t]
                    iyl, iyr, dsn, dn = res[4 + t]
                    west = ixl << LOG2_GRID
                    east = ixr << LOG2_GRID
                    idx[0, j, pl.ds(k * 16, 16)] = west + iyl
                    idx[1, j, pl.ds(k * 16, 16)] = west + iyr
                    idx[2, j, pl.ds(k * 16, 16)] = east + iyr
                    idx[3, j, pl.ds(k * 16, 16)] = east + iyl
                    rden = 1.0 / ((dw + de) * (dn + dsn))
                    wbuf[0, pl.ds(off, 16)] = dn * de * rden
                    wbuf[1, pl.ds(off, 16)] = dsn * de * rden
                    wbuf[2, pl.ds(off, 16)] = dsn * dw * rden
                    wbuf[3, pl.ds(off, 16)] = dn * dw * rden
            for c in range(4):
                pltpu.async_copy(zs_hbm.at[idx.at[c, j]], cbuf.at[c, j], gsem)
            return 0

        lax.fori_loop(0, NB, row, 0)

    def drain_pass2(bs):
        wbuf, idx, cbuf, outb, gsem = bs[2], bs[3], bs[4], bs[5], bs[7]

        def dr(j, _):
            for c in range(4):
                pltpu.make_async_copy(zs_hbm.at[idx.at[c, j]],
                                      cbuf.at[c, j], gsem).wait()
            return 0

        def p2(j, _):
            for k in range(8):
                off = j * 128 + k * 16
                acc = cbuf[0, j, pl.ds(k * 16, 16)] * wbuf[0, pl.ds(off, 16)]
                acc += cbuf[1, j, pl.ds(k * 16, 16)] * wbuf[1, pl.ds(off, 16)]
                acc += cbuf[2, j, pl.ds(k * 16, 16)] * wbuf[2, pl.ds(off, 16)]
                acc += cbuf[3, j, pl.ds(k * 16, 16)] * wbuf[3, pl.ds(off, 16)]
                outb[pl.ds(off, 16)] = acc
            return 0

        lax.fori_loop(0, NB, dr, 0)
        lax.fori_loop(0, NB, p2, 0)

    def out_fire(g, bs):
        pltpu.async_copy(bs[5], out_hbm.at[pl.ds(base + g * B, B)], bs[8])

    def out_wait(bs):
        pltpu.make_async_copy(bs[5], out_hbm.at[pl.ds(base, B)], bs[8]).wait()

    def step(g, cur, nxt):
        wait_fetch(cur)

        @pl.when(g + 1 < NCHUNK)
        def _():
            fetch(g + 1, nxt)

        pass1(cur)

        @pl.when(g >= 1)
        def _():
            @pl.when(g >= 3)
            def _():
                out_wait(nxt)
            drain_pass2(nxt)
            out_fire(g - 1, nxt)

    fetch(0, bufA)

    def pair(g2, _):
        g = g2 * 2
        step(g, bufA, bufB)
        step(g + 1, bufB, bufA)
        return 0

    lax.fori_loop(0, NCHUNK // 2, pair, 0)

    out_wait(bufB)
    drain_pass2(bufB)
    out_fire(NCHUNK - 1, bufB)
    out_wait(bufA)
    out_wait(bufB)


@jax.jit
def _interp(interp_xs, interp_ys, xs, ys, zs_flat):
    mesh = plsc.VectorSubcoreMesh(core_axis_name="c", subcore_axis_name="s")
    rep_types = [pltpu.VMEM((cap * 16,), jnp.float32) for cap in REP_CAPS]
    lohi_types = [pltpu.VMEM((NPATH,), jnp.int32)] * 4
    dbl = [
        pltpu.VMEM((B,), jnp.float32),
        pltpu.VMEM((B,), jnp.float32),
        pltpu.VMEM((4, B), jnp.float32),
        pltpu.VMEM((4, NB, 128), jnp.int32),
        pltpu.VMEM((4, NB, 128), jnp.float32),
        pltpu.VMEM((B,), jnp.float32),
    ]
    fn = pl.kernel(
        _body,
        out_type=jax.ShapeDtypeStruct((NPTS,), jnp.float32),
        mesh=mesh,
        compiler_params=pltpu.CompilerParams(needs_layout_passes=False),
        scratch_types=(
            [pltpu.VMEM((GRID,), jnp.float32),
             pltpu.VMEM((GRID,), jnp.float32)]
            + rep_types + rep_types
            + lohi_types + lohi_types
            + dbl + dbl
            + [pltpu.SemaphoreType.DMA] * 6
        ),
    )
    return fn(interp_xs, interp_ys, xs, ys, zs_flat)


def kernel(interp_xs, interp_ys, xs, ys, zs):
    return _interp(interp_xs, interp_ys, xs, ys, zs.reshape(-1))


# speedup vs baseline: 1728.3067x; 1.0002x over previous
import jax
import jax.numpy as jnp
from jax import lax
from jax.experimental import pallas as pl
from jax.experimental.pallas import tpu as pltpu
from jax.experimental.pallas import tpu_sc as plsc

GRID = 4096
NPTS = 4194304
NC = 2
NS = 16
NW = NC * NS
PTS_PER_TILE = NPTS // NW
B = 2048
NB = B // 128
NCHUNK = PTS_PER_TILE // B
LOG2_GRID = 12
SEARCH_STEPS = 13
REP_LEVELS = 10
TAIL_STEPS = SEARCH_STEPS - REP_LEVELS
NPATH = 1 << REP_LEVELS
REP_CAPS = [max(1 << s, 16) for s in range(REP_LEVELS)]


def _build_axis(table_ref, reps, lo_e, hi_e, lo_o, hi_o):
    lanes = lax.iota(jnp.int32, 16)

    def init(i, _):
        lo_e[pl.ds(i * 16, 16)] = jnp.zeros((16,), jnp.int32)
        hi_e[pl.ds(i * 16, 16)] = jnp.full((16,), GRID, jnp.int32)
        lo_o[pl.ds(i * 16, 16)] = jnp.zeros((16,), jnp.int32)
        hi_o[pl.ds(i * 16, 16)] = jnp.full((16,), GRID, jnp.int32)
        return 0

    lax.fori_loop(0, NPATH // 16, init, 0)

    for s in range(REP_LEVELS):
        src_lo, src_hi = (lo_e, hi_e) if s % 2 == 0 else (lo_o, hi_o)
        dst_lo, dst_hi = (lo_o, hi_o) if s % 2 == 0 else (lo_e, hi_e)
        rep = reps[s]
        nvec = max(1, (1 << s) // 16)

        def bodyb(i, _, rep=rep, src_lo=src_lo, src_hi=src_hi,
                  dst_lo=dst_lo, dst_hi=dst_hi):
            lo = src_lo[pl.ds(i * 16, 16)]
            hi = src_hi[pl.ds(i * 16, 16)]
            mid = (lo + hi) >> 1
            v = plsc.load_gather(table_ref, [mid])
            for pl_ in range(16):
                bc = jnp.broadcast_to(v[pl_], (16,))
                rep[pl.ds(i * 256 + pl_ * 16, 16)] = bc
            p2 = (i * 16 + lanes) << 1
            plsc.store_scatter(dst_lo, [p2], lo)
            plsc.store_scatter(dst_hi, [p2], mid)
            plsc.store_scatter(dst_lo, [p2 + 1], mid + 1)
            plsc.store_scatter(dst_hi, [p2 + 1], hi)
            return 0

        lax.fori_loop(0, nvec, bodyb, 0)


def _axis_dists_multi(axes, vals):
    n = len(vals)
    lanes = lax.iota(jnp.int32, 16)
    p16 = [jnp.zeros((16,), jnp.int32) for _ in range(n)]
    for s in range(REP_LEVELS):
        v = [plsc.load_gather(axes[c][1][s], [p16[c] + lanes])
             for c in range(n)]
        pred = [v[c] < vals[c] for c in range(n)]
        p16 = [(p16[c] << 1) + jnp.where(pred[c], 16, 0) for c in range(n)]
    p = [p16[c] >> 4 for c in range(n)]
    lo = [plsc.load_gather(axes[c][2], [p[c]]) for c in range(n)]
    hi = [plsc.load_gather(axes[c][3], [p[c]]) for c in range(n)]
    for _ in range(TAIL_STEPS):
        mid = [jnp.minimum((lo[c] + hi[c]) >> 1, GRID - 1) for c in range(n)]
        v = [plsc.load_gather(axes[c][0], [mid[c]]) for c in range(n)]
        pred = [v[c] < vals[c] for c in range(n)]
        lo = [jnp.where(pred[c], mid[c] + 1, lo[c]) for c in range(n)]
        hi = [jnp.where(pred[c], hi[c], mid[c]) for c in range(n)]
    r = [jnp.minimum(lo[c], GRID - 1) for c in range(n)]
    l = [jnp.maximum(r[c] - 1, 0) for c in range(n)]
    vl = [plsc.load_gather(axes[c][0], [l[c]]) for c in range(n)]
    vr = [plsc.load_gather(axes[c][0], [r[c]]) for c in range(n)]
    out = []
    for c in range(n):
        dl = jnp.maximum(vals[c] - vl[c], 0.0)
        dr = jnp.maximum(vr[c] - vals[c], 0.0)
        both_zero = (dl == 0.0) & (dr == 0.0)
        dl = jnp.where(both_zero, 1.0, dl)
        dr = jnp.where(both_zero, 1.0, dr)
        out.append((l[c], r[c], dl, dr))
    return out


def _body(xq_hbm, yq_hbm, xs_hbm, ys_hbm, zs_hbm, out_hbm, *refs):
    xs_t, ys_t = refs[0], refs[1]
    repx = list(refs[2:12])
    repy = list(refs[12:22])
    lox_e, hix_e, lox_o, hix_o = refs[22:26]
    loy_e, hiy_e, loy_o, hiy_o = refs[26:30]
    xqA, yqA, wA, idxA, cA, outA = refs[30:36]
    xqB, yqB, wB, idxB, cB, outB = refs[36:42]
    insemA, insemB, gsemA, gsemB, osemA, osemB = refs[42:48]

    wid = lax.axis_index("s") * NC + lax.axis_index("c")
    base = wid * PTS_PER_TILE

    pltpu.sync_copy(xs_hbm, xs_t)
    pltpu.sync_copy(ys_hbm, ys_t)
    _build_axis(xs_t, repx, lox_e, hix_e, lox_o, hix_o)
    _build_axis(ys_t, repy, loy_e, hiy_e, loy_o, hiy_o)
    xaxis = (xs_t, repx, lox_e, hix_e)
    yaxis = (ys_t, repy, loy_e, hiy_e)

    bufA = (xqA, yqA, wA, idxA, cA, outA, insemA, gsemA, osemA)
    bufB = (xqB, yqB, wB, idxB, cB, outB, insemB, gsemB, osemB)

    def fetch(g, bs):
        xq, yq, insem = bs[0], bs[1], bs[6]
        off = base + g * B
        pltpu.async_copy(xq_hbm.at[pl.ds(off, B)], xq, insem)
        pltpu.async_copy(yq_hbm.at[pl.ds(off, B)], yq, insem)

    def wait_fetch(bs):
        xq, yq, insem = bs[0], bs[1], bs[6]
        pltpu.make_async_copy(xq_hbm.at[pl.ds(base, B)], xq, insem).wait()
        pltpu.make_async_copy(yq_hbm.at[pl.ds(base, B)], yq, insem).wait()

    def pass1(bs):
        xq, yq, wbuf, idx, cbuf, gsem = (bs[0], bs[1], bs[2], bs[3], bs[4],
                                         bs[7])

        def row(j, _):
            for h in range(1):
                ks = [8 * h + t for t in range(8)]
                xv = [xq[pl.ds(j * 128 + k * 16, 16)] for k in ks]
                yv = [yq[pl.ds(j * 128 + k * 16, 16)] for k in ks]
                res = _axis_dists_multi([xaxis] * 8 + [yaxis] * 8, xv + yv)
                for t, k in enumerate(ks):
                    off = j * 128 + k * 16
                    ixl, ixr, dw, de = res[t]
                    iyl, iyr, dsn, dn = res[8 + t]
                    west = ixl << LOG2_GRID
                    east = ixr << LOG2_GRID
                    idx[0, j, pl.ds(k * 16, 16)] = west + iyl
                    idx[1, j, pl.ds(k * 16, 16)] = west + iyr
                    idx[2, j, pl.ds(k * 16, 16)] = east + iyr
                    idx[3, j, pl.ds(k * 16, 16)] = east + iyl
                    rden = 1.0 / ((dw + de) * (dn + dsn))
                    wbuf[0, pl.ds(off, 16)] = dn * de * rden
                    wbuf[1, pl.ds(off, 16)] = dsn * de * rden
                    wbuf[2, pl.ds(off, 16)] = dsn * dw * rden
                    wbuf[3, pl.ds(off, 16)] = dn * dw * rden
            for c in range(4):
                pltpu.async_copy(zs_hbm.at[idx.at[c, j]], cbuf.at[c, j], gsem)
            return 0

        lax.fori_loop(0, NB, row, 0)

    def drain_pass2(bs):
        wbuf, idx, cbuf, outb, gsem = bs[2], bs[3], bs[4], bs[5], bs[7]

        def dr(j, _):
            for c in range(4):
                pltpu.make_async_copy(zs_hbm.at[idx.at[c, j]],
                                      cbuf.at[c, j], gsem).wait()
            return 0

        def p2(j, _):
            for k in range(8):
                off = j * 128 + k * 16
                acc = cbuf[0, j, pl.ds(k * 16, 16)] * wbuf[0, pl.ds(off, 16)]
                acc += cbuf[1, j, pl.ds(k * 16, 16)] * wbuf[1, pl.ds(off, 16)]
                acc += cbuf[2, j, pl.ds(k * 16, 16)] * wbuf[2, pl.ds(off, 16)]
                acc += cbuf[3, j, pl.ds(k * 16, 16)] * wbuf[3, pl.ds(off, 16)]
                outb[pl.ds(off, 16)] = acc
            return 0

        lax.fori_loop(0, NB, dr, 0)
        lax.fori_loop(0, NB, p2, 0)

    def out_fire(g, bs):
        pltpu.async_copy(bs[5], out_hbm.at[pl.ds(base + g * B, B)], bs[8])

    def out_wait(bs):
        pltpu.make_async_copy(bs[5], out_hbm.at[pl.ds(base, B)], bs[8]).wait()

    def step(g, cur, nxt):
        wait_fetch(cur)

        @pl.when(g + 1 < NCHUNK)
        def _():
            fetch(g + 1, nxt)

        pass1(cur)

        @pl.when(g >= 1)
        def _():
            @pl.when(g >= 3)
            def _():
                out_wait(nxt)
            drain_pass2(nxt)
            out_fire(g - 1, nxt)

    fetch(0, bufA)

    def pair(g2, _):
        g = g2 * 2
        step(g, bufA, bufB)
        step(g + 1, bufB, bufA)
        return 0

    lax.fori_loop(0, NCHUNK // 2, pair, 0)

    out_wait(bufB)
    drain_pass2(bufB)
    out_fire(NCHUNK - 1, bufB)
    out_wait(bufA)
    out_wait(bufB)


@jax.jit
def _interp(interp_xs, interp_ys, xs, ys, zs_flat):
    mesh = plsc.VectorSubcoreMesh(core_axis_name="c", subcore_axis_name="s")
    rep_types = [pltpu.VMEM((cap * 16,), jnp.float32) for cap in REP_CAPS]
    lohi_types = [pltpu.VMEM((NPATH,), jnp.int32)] * 4
    dbl = [
        pltpu.VMEM((B,), jnp.float32),
        pltpu.VMEM((B,), jnp.float32),
        pltpu.VMEM((4, B), jnp.float32),
        pltpu.VMEM((4, NB, 128), jnp.int32),
        pltpu.VMEM((4, NB, 128), jnp.float32),
        pltpu.VMEM((B,), jnp.float32),
    ]
    fn = pl.kernel(
        _body,
        out_type=jax.ShapeDtypeStruct((NPTS,), jnp.float32),
        mesh=mesh,
        compiler_params=pltpu.CompilerParams(needs_layout_passes=False),
        scratch_types=(
            [pltpu.VMEM((GRID,), jnp.float32),
             pltpu.VMEM((GRID,), jnp.float32)]
            + rep_types + rep_types
            + lohi_types + lohi_types
            + dbl + dbl
            + [pltpu.SemaphoreType.DMA] * 6
        ),
    )
    return fn(interp_xs, interp_ys, xs, ys, zs_flat)


def kernel(interp_xs, interp_ys, xs, ys, zs):
    return _interp(interp_xs, interp_ys, xs, ys, zs.reshape(-1))
